# Initial kernel scaffold; baseline (speedup 1.0000x reference)
import jax
import jax.numpy as jnp
from jax.experimental import pallas as pl


def kernel(n_feats, edge_index, etype, r_feats, num_rels, W_I_w, W_I_b, W_O_w, W_O_b, W_R_w, W_R_b):
    raise NotImplementedError("write your pallas kernel here")


# speedup vs baseline: 6.4420x; 6.4420x over previous
import functools

import jax
import jax.numpy as jnp
from jax import lax
from jax.experimental import pallas as pl
from jax.experimental.pallas import tpu as pltpu
from jax.experimental.pallas import tpu_sc as plsc

_N = 10000
_E = 320000
_D = 128
_R = 200

_NC = 2
_NS = 16
_DH = _D // _NC
_CHUNK = 128
_NCHUNK = _E // _CHUNK
_NPAD = 10240
_RPT = _NPAD // _NS
_CW = 16


def _tables_body(x_ref, w_ref, o_ref):
    o_ref[...] = jnp.dot(x_ref[...], w_ref[0], preferred_element_type=jnp.float32)


def _build_T(n_feats, w_stack):
    return pl.pallas_call(
        _tables_body,
        grid=(2, 10),
        in_specs=[
            pl.BlockSpec((_N // 10, _D), lambda i, j: (j, 0)),
            pl.BlockSpec((1, _D, _D), lambda i, j: (i, 0, 0)),
        ],
        out_specs=pl.BlockSpec((_N // 10, _D), lambda i, j: (i * 10 + j, 0)),
        out_shape=jax.ShapeDtypeStruct((2 * _N, _D), jnp.float32),
    )(n_feats, w_stack)


def _rel_body(r_ref, wI_ref, wO_ref, wR_ref, bI_ref, bO_ref, bR_ref,
              qneg_ref, rout_ref):
    r = r_ref[...]
    rI = jnp.dot(r, wI_ref[...], preferred_element_type=jnp.float32)
    rO = jnp.dot(r, wO_ref[...], preferred_element_type=jnp.float32)
    rR = jnp.dot(r, wR_ref[...], preferred_element_type=jnp.float32)
    rowid = lax.broadcasted_iota(jnp.int32, (_R, _D), 0)
    half = _R // 2
    qneg_ref[...] = jnp.where(rowid < half, bI_ref[...] - rI, bO_ref[...] - rO)
    rout_ref[...] = rR + bR_ref[...]


def _build_rel(r_feats, wI, wO, wR, bI, bO, bR):
    return pl.pallas_call(
        _rel_body,
        out_shape=(
            jax.ShapeDtypeStruct((_R, _D), jnp.float32),
            jax.ShapeDtypeStruct((_R, _D), jnp.float32),
        ),
    )(r_feats, wI, wO, wR, bI.reshape(1, _D), bO.reshape(1, _D),
      bR.reshape(1, _D))


def _gidx_body(src_ref, et_ref, o_ref):
    half = _R // 2
    o_ref[...] = src_ref[...] + jnp.where(et_ref[...] >= half, _N, 0)


def _build_gidx(src2, et2):
    return pl.pallas_call(
        _gidx_body,
        out_shape=jax.ShapeDtypeStruct(src2.shape, jnp.int32),
    )(src2, et2)


def _make_sc_kernel():
    mesh = plsc.VectorSubcoreMesh(core_axis_name="c", subcore_axis_name="s")

    @functools.partial(
        pl.kernel,
        out_type=(
            jax.ShapeDtypeStruct((_NC, _NPAD, _DH), jnp.float32),
            jax.ShapeDtypeStruct((_NC, _NPAD, _CW), jnp.float32),
        ),
        mesh=mesh,
        compiler_params=pltpu.CompilerParams(use_tc_tiling_on_sc=False),
        scratch_types=[
            pltpu.VMEM((_CHUNK,), jnp.int32),
            pltpu.VMEM((_CHUNK,), jnp.int32),
            pltpu.VMEM((_CHUNK,), jnp.int32),
            pltpu.VMEM((_CHUNK, _DH), jnp.float32),
            pltpu.VMEM((_CHUNK, _DH), jnp.float32),
            pltpu.VMEM((_CHUNK, _CW), jnp.float32),
            pltpu.VMEM((_CHUNK, _CW), jnp.float32),
            pltpu.VMEM_SHARED((_NPAD, _DH), jnp.float32),
            pltpu.VMEM_SHARED((_NPAD, _CW), jnp.float32),
            pltpu.SemaphoreType.DMA,
            pltpu.SemaphoreType.DMA,
        ],
    )
    def sc_kernel(T_hbm, qneg_hbm, gidx_hbm, qidx_hbm, dst_hbm,
                  acc_hbm, cnt_hbm,
                  gidx_v, qidx_v, dst_v, rows_v, qrows_v, ones_v, zc_v,
                  acc_s, cnt_s, semi, semg):
        cid = lax.axis_index("c")
        sid = lax.axis_index("s")
        r0 = sid * _RPT

        zero16 = jnp.zeros((16,), jnp.float32)
        one16 = jnp.ones((16,), jnp.float32)

        def _fill(i, carry):
            for c8 in range(_DH // 16):
                rows_v[i, pl.ds(c8 * 16, 16)] = zero16
            zc_v[i, pl.ds(0, _CW)] = zero16
            ones_v[i, pl.ds(0, _CW)] = one16
            return carry

        lax.fori_loop(0, _CHUNK, _fill, 0)

        for b in range(_RPT // _CHUNK):
            off = r0 + b * _CHUNK
            pltpu.sync_copy(rows_v, acc_s.at[pl.ds(off, _CHUNK)])
            pltpu.sync_copy(zc_v, cnt_s.at[pl.ds(off, _CHUNK)])
        plsc.subcore_barrier()

        lo = sid * _NCHUNK // _NS
        hi = (sid + 1) * _NCHUNK // _NS

        def _chunk(c, carry):
            base = pl.multiple_of(c * _CHUNK, _CHUNK)
            ci = pltpu.async_copy(gidx_hbm.at[pl.ds(base, _CHUNK)], gidx_v, semi)
            cq = pltpu.async_copy(qidx_hbm.at[pl.ds(base, _CHUNK)], qidx_v, semi)
            cd = pltpu.async_copy(dst_hbm.at[pl.ds(base, _CHUNK)], dst_v, semi)
            ci.wait()
            cq.wait()
            cd.wait()
            g1 = pltpu.async_copy(T_hbm.at[cid].at[gidx_v], rows_v, semg)
            g2 = pltpu.async_copy(qneg_hbm.at[cid].at[qidx_v], qrows_v, semg)
            g1.wait()
            g2.wait()
            pltpu.sync_copy(rows_v, acc_s.at[dst_v], add=True)
            pltpu.sync_copy(qrows_v, acc_s.at[dst_v], add=True)
            pltpu.sync_copy(ones_v, cnt_s.at[dst_v], add=True)
            return carry

        lax.fori_loop(lo, hi, _chunk, 0)
        plsc.subcore_barrier()

        for b in range(_RPT // _CHUNK):
            off = r0 + b * _CHUNK
            pltpu.sync_copy(acc_s.at[pl.ds(off, _CHUNK)], rows_v)
            pltpu.sync_copy(rows_v, acc_hbm.at[cid, pl.ds(off, _CHUNK)])
            pltpu.sync_copy(cnt_s.at[pl.ds(off, _CHUNK)], zc_v)
            pltpu.sync_copy(zc_v, cnt_hbm.at[cid, pl.ds(off, _CHUNK)])

    return sc_kernel


def _mean_body(acc_ref, cnt_ref, o_ref):
    c = cnt_ref[0]
    cc = jnp.sum(c, axis=-1, keepdims=True) * (1.0 / _CW)
    o_ref[...] = acc_ref[...] / jnp.maximum(cc, 1.0)


def _segment_mean(acc_full, cnt):
    nb = 5
    rb = _NPAD // nb
    return pl.pallas_call(
        _mean_body,
        grid=(nb,),
        in_specs=[
            pl.BlockSpec((rb, _D), lambda i: (i, 0)),
            pl.BlockSpec((1, rb, _CW), lambda i: (0, i, 0)),
        ],
        out_specs=pl.BlockSpec((rb, _D), lambda i: (i, 0)),
        out_shape=jax.ShapeDtypeStruct((_NPAD, _D), jnp.float32),
    )(acc_full, cnt)


def kernel(n_feats, edge_index, etype, r_feats, num_rels,
           W_I_w, W_I_b, W_O_w, W_O_b, W_R_w, W_R_b):
    w_stack = jnp.stack([W_I_w, W_O_w])
    T = _build_T(n_feats, w_stack)
    qneg, r_out = _build_rel(r_feats, W_I_w, W_O_w, W_R_w, W_I_b, W_O_b, W_R_b)

    T_split = jnp.stack([T[:, :_DH], T[:, _DH:]])
    qneg_split = jnp.stack([qneg[:, :_DH], qneg[:, _DH:]])

    src2 = edge_index[0].reshape(_E // _D, _D)
    et2 = etype.reshape(_E // _D, _D)
    gidx = _build_gidx(src2, et2).reshape(_E)

    sc = _make_sc_kernel()
    acc, cnt = sc(T_split, qneg_split, gidx, etype, edge_index[1])

    acc_full = jnp.concatenate([acc[0], acc[1]], axis=1)
    n_out = _segment_mean(acc_full, cnt)[:_N]
    return (n_out, r_out)
